# baseline (speedup 1.0000x reference)
import functools

import jax
import jax.numpy as jnp
from jax import lax
from jax.experimental import pallas as pl
from jax.experimental.pallas import tpu as pltpu
from jax.experimental.pallas import tpu_sc as plsc

N_EXPERT = 8
TOP_K = 2
N_EMBD = 2048
D_FF = 4096
N_TOK = 2048

RB = 128
NB = N_TOK * TOP_K // RB + N_EXPERT
NP = NB * RB
FB = 256
NF = D_FF // FB

NW = 32

NEG_BIG = -1e30


def _gate_body(x_ref, wg_ref, idx_ref, prb_ref, xb_ref):
    s = lax.dot_general(
        x_ref[...], wg_ref[...],
        dimension_numbers=(((1,), (1,)), ((), ())),
        preferred_element_type=jnp.float32,
    )
    n, e = s.shape
    col = lax.broadcasted_iota(jnp.int32, (n, e), 1)
    m1 = jnp.max(s, axis=1, keepdims=True)
    i1 = jnp.min(jnp.where(s == m1, col, e), axis=1, keepdims=True)
    s2 = jnp.where(col == i1, NEG_BIG, s)
    m2 = jnp.max(s2, axis=1, keepdims=True)
    i2 = jnp.min(jnp.where(s2 == m2, col, e), axis=1, keepdims=True)
    t = jnp.exp(m2 - m1)
    p1 = 1.0 / (1.0 + t)
    p2 = 1.0 - p1
    idx_ref[...] = jnp.where(col == 0, i1, jnp.where(col == 1, i2, 0))
    prb_ref[...] = jnp.where(col == 0, p1, jnp.where(col == 1, p2, 0.0))
    xb_ref[...] = x_ref[...].astype(jnp.bfloat16)


def _gate(x_flat, Wg):
    n = x_flat.shape[0]
    return pl.pallas_call(
        _gate_body,
        out_shape=(
            jax.ShapeDtypeStruct((n, N_EXPERT), jnp.int32),
            jax.ShapeDtypeStruct((n, N_EXPERT), jnp.float32),
            jax.ShapeDtypeStruct((n, N_EMBD), jnp.bfloat16),
        ),
    )(x_flat, Wg)


def _routing_metadata(idx8, prb8):
    n = idx8.shape[0]
    pair_e = jnp.concatenate([idx8[:, 0], idx8[:, 1]])
    pair_t = jnp.concatenate([jnp.arange(n, dtype=jnp.int32)] * 2)
    pair_p = jnp.concatenate([prb8[:, 0], prb8[:, 1]])

    onehot = (pair_e[:, None] == jnp.arange(N_EXPERT, dtype=jnp.int32)[None, :])
    onehot = onehot.astype(jnp.int32)
    excl = lax.associative_scan(jnp.add, onehot, axis=0) - onehot
    rank = jnp.take_along_axis(excl, pair_e[:, None], axis=1)[:, 0]
    counts = jnp.sum(onehot, axis=0)
    padded = ((counts + RB - 1) // RB) * RB
    offs = jnp.concatenate([jnp.zeros((1,), jnp.int32),
                            jnp.cumsum(padded)[:-1]])
    ppos = offs[pair_e] + rank

    src = jnp.zeros((NP,), jnp.int32).at[ppos].set(pair_t)
    prob = jnp.zeros((NP,), jnp.float32).at[ppos].set(pair_p)
    blk_off = offs // RB
    blk_e = (jnp.searchsorted(blk_off, jnp.arange(NB, dtype=jnp.int32),
                              side="right") - 1).astype(jnp.int32)
    posA = ppos[:n].astype(jnp.int32)
    posB = ppos[n:].astype(jnp.int32)
    return src, prob, blk_e, posA, posB


C_I32 = N_EMBD // 2


def _sc_dispatch(idx, table):
    per_w = NP // NW
    ch = 16
    mesh = plsc.VectorSubcoreMesh(core_axis_name="c", subcore_axis_name="s")

    @functools.partial(
        pl.kernel,
        out_type=jax.ShapeDtypeStruct((NP, C_I32), jnp.int32),
        mesh=mesh,
        scratch_types=[
            pltpu.VMEM((per_w,), jnp.int32),
            pltpu.VMEM((ch, C_I32), jnp.int32),
            pltpu.VMEM((ch, C_I32), jnp.int32),
            pltpu.SemaphoreType.DMA,
            pltpu.SemaphoreType.DMA,
        ],
    )
    def k(idx_hbm, table_hbm, out_hbm, idx_v, r0, r1, sem0, sem1):
        wid = lax.axis_index("s") * 2 + lax.axis_index("c")
        base = wid * per_w
        pltpu.sync_copy(idx_hbm.at[pl.ds(base, per_w)], idx_v)

        n_ch = per_w // ch
        bufs, sems = (r0, r1), (sem0, sem1)
        cps = [None] * n_ch
        cps[0] = pltpu.async_copy(
            table_hbm.at[idx_v.at[pl.ds(0, ch)]], r0, sem0)
        for c in range(n_ch):
            if c + 1 < n_ch:
                cps[c + 1] = pltpu.async_copy(
                    table_hbm.at[idx_v.at[pl.ds((c + 1) * ch, ch)]],
                    bufs[(c + 1) % 2], sems[(c + 1) % 2])
            cps[c].wait()
            pltpu.sync_copy(bufs[c % 2], out_hbm.at[pl.ds(base + c * ch, ch)])

    return k(idx, table)


def _sc_combine(posA, posB, table):
    per_w = N_TOK // NW
    ch = 16
    mesh = plsc.VectorSubcoreMesh(core_axis_name="c", subcore_axis_name="s")

    @functools.partial(
        pl.kernel,
        out_type=(
            jax.ShapeDtypeStruct((N_TOK, N_EMBD), jnp.float32),
            jax.ShapeDtypeStruct((N_TOK, N_EMBD), jnp.float32),
        ),
        mesh=mesh,
        scratch_types=[
            pltpu.VMEM((per_w,), jnp.int32),
            pltpu.VMEM((per_w,), jnp.int32),
            pltpu.VMEM((ch, N_EMBD), jnp.float32),
            pltpu.VMEM((ch, N_EMBD), jnp.float32),
            pltpu.SemaphoreType.DMA,
            pltpu.SemaphoreType.DMA,
        ],
    )
    def k(pa_hbm, pb_hbm, table_hbm, ga_hbm, gb_hbm,
          pa_v, pb_v, r0, r1, sem0, sem1):
        wid = lax.axis_index("s") * 2 + lax.axis_index("c")
        base = wid * per_w
        pltpu.sync_copy(pa_hbm.at[pl.ds(base, per_w)], pa_v)
        pltpu.sync_copy(pb_hbm.at[pl.ds(base, per_w)], pb_v)

        n_ch = per_w // ch
        plan = []
        for c in range(n_ch):
            plan.append((pa_v, ga_hbm, c))
            plan.append((pb_v, gb_hbm, c))
        bufs, sems = (r0, r1), (sem0, sem1)
        cps = [None] * len(plan)

        def start(i):
            iv, _, c = plan[i]
            return pltpu.async_copy(
                table_hbm.at[iv.at[pl.ds(c * ch, ch)]],
                bufs[i % 2], sems[i % 2])

        cps[0] = start(0)
        for i in range(len(plan)):
            if i + 1 < len(plan):
                cps[i + 1] = start(i + 1)
            cps[i].wait()
            _, oh, c = plan[i]
            pltpu.sync_copy(bufs[i % 2], oh.at[pl.ds(base + c * ch, ch)])

    return k(posA, posB, table)


def _add_body(a_ref, b_ref, o_ref):
    o_ref[...] = a_ref[...] + b_ref[...]


def _add(a, b):
    n = a.shape[0]
    blk = 512
    return pl.pallas_call(
        _add_body,
        grid=(n // blk,),
        in_specs=[pl.BlockSpec((blk, N_EMBD), lambda i: (i, 0)),
                  pl.BlockSpec((blk, N_EMBD), lambda i: (i, 0))],
        out_specs=pl.BlockSpec((blk, N_EMBD), lambda i: (i, 0)),
        out_shape=jax.ShapeDtypeStruct((n, N_EMBD), jnp.float32),
    )(a, b)


def _mlp1_body(be_ref, xb_ref, w1_ref, prb_ref, h_ref, w1c_ref):
    b = pl.program_id(1)
    same = jnp.logical_and(b > 0, be_ref[b] == be_ref[jnp.maximum(b - 1, 0)])

    @pl.when(jnp.logical_not(same))
    def _():
        w1c_ref[...] = w1_ref[0].astype(jnp.bfloat16)

    xb = xb_ref[pl.ds(b * RB, RB), :]
    h = lax.dot_general(
        xb, w1c_ref[...],
        dimension_numbers=(((1,), (1,)), ((), ())),
        preferred_element_type=jnp.float32,
    )
    h = h * (1.0 / (1.0 + jnp.exp(-h)))
    h = h * prb_ref[:, 0:1]
    h_ref[...] = h.astype(jnp.bfloat16)


def _mlp1(xb, W1, prob8, blk_e):
    grid_spec = pltpu.PrefetchScalarGridSpec(
        num_scalar_prefetch=1,
        grid=(NF, NB),
        in_specs=[
            pl.BlockSpec((NP, N_EMBD), lambda f, b, be: (0, 0)),
            pl.BlockSpec((1, FB, N_EMBD), lambda f, b, be: (be[b], f, 0)),
            pl.BlockSpec((RB, N_EXPERT), lambda f, b, be: (b, 0)),
        ],
        out_specs=pl.BlockSpec((RB, FB), lambda f, b, be: (b, f)),
        scratch_shapes=[
            pltpu.VMEM((FB, N_EMBD), jnp.bfloat16),
        ],
    )
    return pl.pallas_call(
        _mlp1_body,
        grid_spec=grid_spec,
        out_shape=jax.ShapeDtypeStruct((NP, D_FF), jnp.bfloat16),
        compiler_params=pltpu.CompilerParams(
            dimension_semantics=("arbitrary", "arbitrary"),
            vmem_limit_bytes=100 * 1024 * 1024,
        ),
    )(blk_e, xb, W1, prob8)


def _mlp2_body(be_ref, h_ref, w2_ref, o_ref, w2c_ref):
    f = pl.program_id(0)
    b = pl.program_id(1)
    same = jnp.logical_and(b > 0, be_ref[b] == be_ref[jnp.maximum(b - 1, 0)])

    @pl.when(jnp.logical_not(same))
    def _():
        w2c_ref[...] = w2_ref[0].astype(jnp.bfloat16)

    acc = lax.dot_general(
        h_ref[...], w2c_ref[...],
        dimension_numbers=(((1,), (1,)), ((), ())),
        preferred_element_type=jnp.float32,
    )

    @pl.when(f == 0)
    def _():
        o_ref[pl.ds(b * RB, RB), :] = acc

    @pl.when(f != 0)
    def _():
        o_ref[pl.ds(b * RB, RB), :] += acc


def _mlp2(h, W2, blk_e):
    grid_spec = pltpu.PrefetchScalarGridSpec(
        num_scalar_prefetch=1,
        grid=(NF, NB),
        in_specs=[
            pl.BlockSpec((RB, FB), lambda f, b, be: (b, f)),
            pl.BlockSpec((1, N_EMBD, FB), lambda f, b, be: (be[b], 0, f)),
        ],
        out_specs=pl.BlockSpec((NP, N_EMBD), lambda f, b, be: (0, 0)),
        scratch_shapes=[
            pltpu.VMEM((N_EMBD, FB), jnp.bfloat16),
        ],
    )
    return pl.pallas_call(
        _mlp2_body,
        grid_spec=grid_spec,
        out_shape=jax.ShapeDtypeStruct((NP, N_EMBD), jnp.float32),
        compiler_params=pltpu.CompilerParams(
            dimension_semantics=("arbitrary", "arbitrary"),
            vmem_limit_bytes=100 * 1024 * 1024,
        ),
    )(blk_e, h, W2)


@jax.jit
def kernel(x, Wg, W1, W2):
    B, T, C = x.shape
    x_flat = x.reshape(-1, C)
    idx8, prb8, x_bf = _gate(x_flat, Wg)
    src, prob, blk_e, posA, posB = _routing_metadata(idx8, prb8)
    x_i32 = lax.bitcast_convert_type(
        x_bf.reshape(-1, C_I32, 2), jnp.int32)
    xb_i32 = _sc_dispatch(src, x_i32)
    xb = lax.bitcast_convert_type(
        xb_i32, jnp.bfloat16).reshape(NP, N_EMBD)
    prob8 = jnp.broadcast_to(prob[:, None], (NP, N_EXPERT))
    h = _mlp1(xb, W1, prob8, blk_e)
    outs = _mlp2(h, W2, blk_e)
    gA, gB = _sc_combine(posA, posB, outs)
    y = _add(gA, gB)
    return y.reshape(B, T, C)


# speedup vs baseline: 1.7256x; 1.7256x over previous
import functools

import jax
import jax.numpy as jnp
from jax import lax
from jax.experimental import pallas as pl
from jax.experimental.pallas import tpu as pltpu
from jax.experimental.pallas import tpu_sc as plsc

N_EXPERT = 8
TOP_K = 2
N_EMBD = 2048
D_FF = 4096
N_TOK = 2048

RB = 128
NB = N_TOK * TOP_K // RB + N_EXPERT
NP = NB * RB
FB = 512
NF = D_FF // FB

NW = 32

NEG_BIG = -1e30


def _gate_body(x_ref, wg_ref, idx_ref, prb_ref):
    s = lax.dot_general(
        x_ref[...], wg_ref[...],
        dimension_numbers=(((1,), (1,)), ((), ())),
        preferred_element_type=jnp.float32,
    )
    n, e = s.shape
    col = lax.broadcasted_iota(jnp.int32, (n, e), 1)
    m1 = jnp.max(s, axis=1, keepdims=True)
    i1 = jnp.min(jnp.where(s == m1, col, e), axis=1, keepdims=True)
    s2 = jnp.where(col == i1, NEG_BIG, s)
    m2 = jnp.max(s2, axis=1, keepdims=True)
    i2 = jnp.min(jnp.where(s2 == m2, col, e), axis=1, keepdims=True)
    t = jnp.exp(m2 - m1)
    p1 = 1.0 / (1.0 + t)
    p2 = 1.0 - p1
    idx_ref[...] = jnp.where(col == 0, i1, jnp.where(col == 1, i2, 0))
    prb_ref[...] = jnp.where(col == 0, p1, jnp.where(col == 1, p2, 0.0))


def _gate(x_flat, Wg):
    n = x_flat.shape[0]
    return pl.pallas_call(
        _gate_body,
        out_shape=(
            jax.ShapeDtypeStruct((n, N_EXPERT), jnp.int32),
            jax.ShapeDtypeStruct((n, N_EXPERT), jnp.float32),
        ),
    )(x_flat, Wg)


def _routing_metadata(idx8):
    n = idx8.shape[0]
    pair_e = jnp.concatenate([idx8[:, 0], idx8[:, 1]])
    pair_t = jnp.concatenate([jnp.arange(n, dtype=jnp.int32)] * 2)

    onehot = (pair_e[:, None] == jnp.arange(N_EXPERT, dtype=jnp.int32)[None, :])
    onehot = onehot.astype(jnp.int32)
    excl = lax.associative_scan(jnp.add, onehot, axis=0) - onehot
    rank = jnp.take_along_axis(excl, pair_e[:, None], axis=1)[:, 0]
    counts = jnp.sum(onehot, axis=0)
    padded = ((counts + RB - 1) // RB) * RB
    offs = jnp.concatenate([jnp.zeros((1,), jnp.int32),
                            jnp.cumsum(padded)[:-1]])
    ppos = offs[pair_e] + rank

    src = jnp.zeros((NP,), jnp.int32).at[ppos].set(pair_t)
    blk_off = offs // RB
    blk_e = (jnp.searchsorted(blk_off, jnp.arange(NB, dtype=jnp.int32),
                              side="right") - 1).astype(jnp.int32)
    posA = ppos[:n].astype(jnp.int32)
    posB = ppos[n:].astype(jnp.int32)
    return src, blk_e, posA, posB


def _sc_dispatch(idx, table):
    per_w = NP // NW
    ch = 16
    mesh = plsc.VectorSubcoreMesh(core_axis_name="c", subcore_axis_name="s")

    @functools.partial(
        pl.kernel,
        out_type=jax.ShapeDtypeStruct((NP, N_EMBD), jnp.float32),
        mesh=mesh,
        scratch_types=[
            pltpu.VMEM((per_w,), jnp.int32),
            pltpu.VMEM((ch, N_EMBD), jnp.float32),
            pltpu.VMEM((ch, N_EMBD), jnp.float32),
            pltpu.SemaphoreType.DMA,
            pltpu.SemaphoreType.DMA,
        ],
    )
    def k(idx_hbm, table_hbm, out_hbm, idx_v, r0, r1, sem0, sem1):
        wid = lax.axis_index("s") * 2 + lax.axis_index("c")
        base = wid * per_w
        pltpu.sync_copy(idx_hbm.at[pl.ds(base, per_w)], idx_v)

        n_ch = per_w // ch
        bufs, sems = (r0, r1), (sem0, sem1)
        cps = [None] * n_ch
        cps[0] = pltpu.async_copy(
            table_hbm.at[idx_v.at[pl.ds(0, ch)]], r0, sem0)
        for c in range(n_ch):
            if c + 1 < n_ch:
                cps[c + 1] = pltpu.async_copy(
                    table_hbm.at[idx_v.at[pl.ds((c + 1) * ch, ch)]],
                    bufs[(c + 1) % 2], sems[(c + 1) % 2])
            cps[c].wait()
            pltpu.sync_copy(bufs[c % 2], out_hbm.at[pl.ds(base + c * ch, ch)])

    return k(idx, table)


def _sc_combine(posA, posB, table):
    per_w = N_TOK // NW
    ch = 16
    mesh = plsc.VectorSubcoreMesh(core_axis_name="c", subcore_axis_name="s")

    @functools.partial(
        pl.kernel,
        out_type=(
            jax.ShapeDtypeStruct((N_TOK, N_EMBD), jnp.float32),
            jax.ShapeDtypeStruct((N_TOK, N_EMBD), jnp.float32),
        ),
        mesh=mesh,
        scratch_types=[
            pltpu.VMEM((per_w,), jnp.int32),
            pltpu.VMEM((per_w,), jnp.int32),
            pltpu.VMEM((ch, N_EMBD), jnp.float32),
            pltpu.VMEM((ch, N_EMBD), jnp.float32),
            pltpu.SemaphoreType.DMA,
            pltpu.SemaphoreType.DMA,
        ],
    )
    def k(pa_hbm, pb_hbm, table_hbm, ga_hbm, gb_hbm,
          pa_v, pb_v, r0, r1, sem0, sem1):
        wid = lax.axis_index("s") * 2 + lax.axis_index("c")
        base = wid * per_w
        pltpu.sync_copy(pa_hbm.at[pl.ds(base, per_w)], pa_v)
        pltpu.sync_copy(pb_hbm.at[pl.ds(base, per_w)], pb_v)

        n_ch = per_w // ch
        plan = []
        for c in range(n_ch):
            plan.append((pa_v, ga_hbm, c))
            plan.append((pb_v, gb_hbm, c))
        bufs, sems = (r0, r1), (sem0, sem1)
        cps = [None] * len(plan)

        def start(i):
            iv, _, c = plan[i]
            return pltpu.async_copy(
                table_hbm.at[iv.at[pl.ds(c * ch, ch)]],
                bufs[i % 2], sems[i % 2])

        cps[0] = start(0)
        for i in range(len(plan)):
            if i + 1 < len(plan):
                cps[i + 1] = start(i + 1)
            cps[i].wait()
            _, oh, c = plan[i]
            pltpu.sync_copy(bufs[i % 2], oh.at[pl.ds(base + c * ch, ch)])

    return k(posA, posB, table)


def _cast_body(x_ref, o_ref):
    o_ref[...] = x_ref[...].astype(jnp.bfloat16)


def _cast_bf16(x):
    n = x.shape[0]
    blk = 512
    return pl.pallas_call(
        _cast_body,
        grid=(n // blk,),
        in_specs=[pl.BlockSpec((blk, N_EMBD), lambda i: (i, 0))],
        out_specs=pl.BlockSpec((blk, N_EMBD), lambda i: (i, 0)),
        out_shape=jax.ShapeDtypeStruct((n, N_EMBD), jnp.bfloat16),
    )(x)


def _wadd_body(a_ref, b_ref, p_ref, o_ref):
    pa = p_ref[:, 0:1]
    pb = p_ref[:, 1:2]
    o_ref[...] = pa * a_ref[...] + pb * b_ref[...]


def _wadd(a, b, prb8):
    n = a.shape[0]
    blk = 512
    return pl.pallas_call(
        _wadd_body,
        grid=(n // blk,),
        in_specs=[pl.BlockSpec((blk, N_EMBD), lambda i: (i, 0)),
                  pl.BlockSpec((blk, N_EMBD), lambda i: (i, 0)),
                  pl.BlockSpec((blk, N_EXPERT), lambda i: (i, 0))],
        out_specs=pl.BlockSpec((blk, N_EMBD), lambda i: (i, 0)),
        out_shape=jax.ShapeDtypeStruct((n, N_EMBD), jnp.float32),
    )(a, b, prb8)


def _mlp_body(be_ref, xb_ref, w1_ref, w2_ref, o_ref, w1c_ref, w2c_ref):
    f = pl.program_id(0)
    b = pl.program_id(1)

    same = jnp.logical_and(b > 0, be_ref[b] == be_ref[jnp.maximum(b - 1, 0)])

    @pl.when(jnp.logical_not(same))
    def _():
        w1c_ref[...] = w1_ref[0].astype(jnp.bfloat16)
        w2c_ref[...] = w2_ref[0].astype(jnp.bfloat16)

    xb = xb_ref[...]
    h = lax.dot_general(
        xb, w1c_ref[...],
        dimension_numbers=(((1,), (1,)), ((), ())),
        preferred_element_type=jnp.float32,
    )
    h = h * (1.0 / (1.0 + jnp.exp(-h)))
    acc = lax.dot_general(
        h.astype(jnp.bfloat16), w2c_ref[...],
        dimension_numbers=(((1,), (1,)), ((), ())),
        preferred_element_type=jnp.float32,
    )

    @pl.when(f == 0)
    def _():
        o_ref[pl.ds(b * RB, RB), :] = acc

    @pl.when(f != 0)
    def _():
        o_ref[pl.ds(b * RB, RB), :] += acc


def _mlp(xb, W1, W2, blk_e):
    grid_spec = pltpu.PrefetchScalarGridSpec(
        num_scalar_prefetch=1,
        grid=(NF, NB),
        in_specs=[
            pl.BlockSpec((RB, N_EMBD), lambda f, b, be: (b, 0)),
            pl.BlockSpec((1, FB, N_EMBD), lambda f, b, be: (be[b], f, 0)),
            pl.BlockSpec((1, N_EMBD, FB), lambda f, b, be: (be[b], 0, f)),
        ],
        out_specs=pl.BlockSpec((NP, N_EMBD), lambda f, b, be: (0, 0)),
        scratch_shapes=[
            pltpu.VMEM((FB, N_EMBD), jnp.bfloat16),
            pltpu.VMEM((N_EMBD, FB), jnp.bfloat16),
        ],
    )
    return pl.pallas_call(
        _mlp_body,
        grid_spec=grid_spec,
        out_shape=jax.ShapeDtypeStruct((NP, N_EMBD), jnp.float32),
        compiler_params=pltpu.CompilerParams(
            dimension_semantics=("arbitrary", "arbitrary"),
            vmem_limit_bytes=100 * 1024 * 1024,
        ),
    )(blk_e, xb, W1, W2)


@jax.jit
def kernel(x, Wg, W1, W2):
    B, T, C = x.shape
    x_flat = x.reshape(-1, C)
    idx8, prb8 = _gate(x_flat, Wg)
    src, blk_e, posA, posB = _routing_metadata(idx8)
    xs = _sc_dispatch(src, x_flat)
    xb = _cast_bf16(xs)
    outs = _mlp(xb, W1, W2, blk_e)
    gA, gB = _sc_combine(posA, posB, outs)
    y = _wadd(gA, gB, prb8)
    return y.reshape(B, T, C)
